# baseline (speedup 1.0000x reference)
import functools

import jax
import jax.numpy as jnp
from jax import lax
from jax.experimental import pallas as pl
from jax.experimental.pallas import tpu as pltpu
from jax.experimental.pallas import tpu_sc as plsc

NC = 2
NS = 16
NW = NC * NS
CHUNK = 128
NBUF = 2
CORE0_FRAC = 0.28


def _sc_aggregate(featpad, src, dst, zrow, n_pad, dw, ch0, ch1, rt):
  mesh = plsc.VectorSubcoreMesh(core_axis_name="c", subcore_axis_name="s")

  @functools.partial(
      pl.kernel,
      mesh=mesh,
      compiler_params=pltpu.CompilerParams(use_tc_tiling_on_sc=False),
      out_type=jax.ShapeDtypeStruct((NC, n_pad, dw), jnp.float32),
      scratch_types=[
          pltpu.VMEM((NBUF, CHUNK), jnp.int32),
          pltpu.VMEM((NBUF, CHUNK), jnp.int32),
          pltpu.VMEM((NBUF, CHUNK, dw), jnp.float32),
          pltpu.VMEM_SHARED((n_pad, dw), jnp.float32),
          pltpu.SemaphoreType.DMA((NBUF,)),
      ],
  )
  def sc_agg(fp_hbm, src_hbm, dst_hbm, z_hbm, out_hbm, sidx, didx, rows,
             acc, sem):
    c = lax.axis_index("c")
    s = lax.axis_index("s")
    row0 = s * rt
    for j in range(rt // CHUNK):
      pltpu.sync_copy(z_hbm, acc.at[pl.ds(row0 + j * CHUNK, CHUNK)])
    plsc.subcore_barrier()

    start_chunk = jnp.where(c == 0, s * ch0, NS * ch0 + s * ch1)
    n_outer = jnp.where(c == 0, ch0 // NBUF, ch1 // NBUF)
    ebase = start_chunk * CHUNK

    def gather_cp(b):
      return pltpu.make_async_copy(fp_hbm.at[sidx.at[b]], rows.at[b],
                                   sem.at[b])

    def prefetch(j, b):
      pltpu.sync_copy(src_hbm.at[pl.ds(ebase + j * CHUNK, CHUNK)],
                      sidx.at[b])
      pltpu.sync_copy(dst_hbm.at[pl.ds(ebase + j * CHUNK, CHUNK)],
                      didx.at[b])
      gather_cp(b).start()

    for b in range(NBUF):
      prefetch(b, b)

    def outer(i, carry):
      for b in range(NBUF):
        j = i * NBUF + b
        gather_cp(b).wait()
        pltpu.sync_copy(rows.at[b], acc.at[didx.at[b]], add=True)
        prefetch(j + NBUF, b)
      return carry

    lax.fori_loop(0, n_outer, outer, 0)
    for b in range(NBUF):
      gather_cp(b).wait()
    plsc.subcore_barrier()
    for j in range(rt // CHUNK):
      r = row0 + j * CHUNK
      pltpu.sync_copy(acc.at[pl.ds(r, CHUNK)], out_hbm.at[c, pl.ds(r, CHUNK)])

  return sc_agg(featpad, src, dst, zrow)


def _tc_body(eps_ref, acc_ref, feat_ref, w1_ref, b1_ref, w2_ref, b2_ref,
             out_ref):
  d = feat_ref.shape[1]
  s = acc_ref[0] + acc_ref[1]
  agg_sum = s[:, :d]
  deg = jnp.maximum(s[:, d:d + 1], 1.0)
  agg = agg_sum / deg
  f = feat_ref[...]
  rst = (1.0 + eps_ref[0, 0]) * f + agg
  z = jnp.dot(rst, w1_ref[...], preferred_element_type=jnp.float32)
  z = jnp.maximum(z + b1_ref[...], 0.0)
  z = jnp.dot(z, w2_ref[...], preferred_element_type=jnp.float32)
  z = jnp.maximum(z + b2_ref[...], 0.0)
  out_ref[...] = z + f


def kernel(feat, edge_index, eps, W1, b1, W2, b2):
  n, d = feat.shape
  e = edge_index.shape[1]
  dw = d + 16

  t_chunks = -(-e // (NW * CHUNK * NBUF)) * NW * NBUF
  e_pad = t_chunks * CHUNK
  ch0 = int(round(t_chunks * CORE0_FRAC / (NS * NBUF))) * NBUF
  ch1 = (t_chunks - NS * ch0) // NS
  assert ch1 % NBUF == 0 and ch1 > 0
  rt = -(-(n + 1) // (NS * CHUNK)) * CHUNK
  n_pad = NS * rt

  src = edge_index[0].astype(jnp.int32)
  dst = edge_index[1].astype(jnp.int32)
  pad_idx = jnp.full((e_pad - e + NBUF * CHUNK,), n, dtype=jnp.int32)
  src = jnp.concatenate([src, pad_idx])
  dst = jnp.concatenate([dst, pad_idx])

  featpad = jnp.concatenate(
      [feat, jnp.ones((n, 1), jnp.float32),
       jnp.zeros((n, dw - d - 1), jnp.float32)], axis=1)
  featpad = jnp.pad(featpad, ((0, n_pad - n), (0, 0)))
  zrow = jnp.zeros((CHUNK, dw), jnp.float32)

  acc = _sc_aggregate(featpad, src, dst, zrow, n_pad, dw, ch0, ch1, rt)

  rows = 1024
  grid = -(-n // rows)
  out = pl.pallas_call(
      _tc_body,
      grid=(grid,),
      in_specs=[
          pl.BlockSpec(memory_space=pltpu.SMEM),
          pl.BlockSpec((NC, rows, dw), lambda i: (0, i, 0)),
          pl.BlockSpec((rows, d), lambda i: (i, 0)),
          pl.BlockSpec((d, d), lambda i: (0, 0)),
          pl.BlockSpec((1, d), lambda i: (0, 0)),
          pl.BlockSpec((d, d), lambda i: (0, 0)),
          pl.BlockSpec((1, d), lambda i: (0, 0)),
      ],
      out_specs=pl.BlockSpec((rows, d), lambda i: (i, 0)),
      out_shape=jax.ShapeDtypeStruct((n, d), jnp.float32),
  )(jnp.asarray(eps, jnp.float32).reshape(1, 1), acc, feat, W1,
    b1.reshape(1, d), W2, b2.reshape(1, d))
  return out


# speedup vs baseline: 1.0871x; 1.0871x over previous
import functools

import jax
import jax.numpy as jnp
from jax import lax
from jax.experimental import pallas as pl
from jax.experimental.pallas import tpu as pltpu
from jax.experimental.pallas import tpu_sc as plsc

NC = 2
NS = 16
NW = NC * NS
CHUNK = 128
NBUF = 2
CORE0_FRAC = 0.70


def _sc_aggregate(featpad, src, dst, zrow, n_pad, dw, ch0, ch1, rt):
  mesh = plsc.VectorSubcoreMesh(core_axis_name="c", subcore_axis_name="s")

  @functools.partial(
      pl.kernel,
      mesh=mesh,
      compiler_params=pltpu.CompilerParams(use_tc_tiling_on_sc=False),
      out_type=jax.ShapeDtypeStruct((NC, n_pad, dw), jnp.float32),
      scratch_types=[
          pltpu.VMEM((NBUF, CHUNK), jnp.int32),
          pltpu.VMEM((NBUF, CHUNK), jnp.int32),
          pltpu.VMEM((NBUF, CHUNK, dw), jnp.float32),
          pltpu.VMEM_SHARED((n_pad, dw), jnp.float32),
          pltpu.SemaphoreType.DMA((NBUF,)),
      ],
  )
  def sc_agg(fp_hbm, src_hbm, dst_hbm, z_hbm, out_hbm, sidx, didx, rows,
             acc, sem):
    c = lax.axis_index("c")
    s = lax.axis_index("s")
    row0 = s * rt
    for j in range(rt // CHUNK):
      pltpu.sync_copy(z_hbm, acc.at[pl.ds(row0 + j * CHUNK, CHUNK)])
    plsc.subcore_barrier()

    start_chunk = jnp.where(c == 0, s * ch0, NS * ch0 + s * ch1)
    n_outer = jnp.where(c == 0, ch0 // NBUF, ch1 // NBUF)
    ebase = start_chunk * CHUNK

    def gather_cp(b):
      return pltpu.make_async_copy(fp_hbm.at[sidx.at[b]], rows.at[b],
                                   sem.at[b])

    def prefetch(j, b):
      pltpu.sync_copy(src_hbm.at[pl.ds(ebase + j * CHUNK, CHUNK)],
                      sidx.at[b])
      pltpu.sync_copy(dst_hbm.at[pl.ds(ebase + j * CHUNK, CHUNK)],
                      didx.at[b])
      gather_cp(b).start()

    for b in range(NBUF):
      prefetch(b, b)

    def outer(i, carry):
      for b in range(NBUF):
        j = i * NBUF + b
        gather_cp(b).wait()
        pltpu.sync_copy(rows.at[b], acc.at[didx.at[b]], add=True)
        prefetch(j + NBUF, b)
      return carry

    lax.fori_loop(0, n_outer, outer, 0)
    for b in range(NBUF):
      gather_cp(b).wait()
    plsc.subcore_barrier()
    for j in range(rt // CHUNK):
      r = row0 + j * CHUNK
      pltpu.sync_copy(acc.at[pl.ds(r, CHUNK)], out_hbm.at[c, pl.ds(r, CHUNK)])

  return sc_agg(featpad, src, dst, zrow)


def _tc_body(eps_ref, acc_ref, feat_ref, w1_ref, b1_ref, w2_ref, b2_ref,
             out_ref):
  d = feat_ref.shape[1]
  s = acc_ref[0] + acc_ref[1]
  agg_sum = s[:, :d]
  deg = jnp.maximum(s[:, d:d + 1], 1.0)
  agg = agg_sum / deg
  f = feat_ref[...]
  rst = (1.0 + eps_ref[0, 0]) * f + agg
  z = jnp.dot(rst, w1_ref[...], preferred_element_type=jnp.float32)
  z = jnp.maximum(z + b1_ref[...], 0.0)
  z = jnp.dot(z, w2_ref[...], preferred_element_type=jnp.float32)
  z = jnp.maximum(z + b2_ref[...], 0.0)
  out_ref[...] = z + f


def kernel(feat, edge_index, eps, W1, b1, W2, b2):
  n, d = feat.shape
  e = edge_index.shape[1]
  dw = d + 16

  t_chunks = -(-e // (NW * CHUNK * NBUF)) * NW * NBUF
  e_pad = t_chunks * CHUNK
  ch0 = int(round(t_chunks * CORE0_FRAC / (NS * NBUF))) * NBUF
  ch1 = (t_chunks - NS * ch0) // NS
  assert ch1 % NBUF == 0 and ch1 > 0
  rt = -(-(n + 1) // (NS * CHUNK)) * CHUNK
  n_pad = NS * rt

  src = edge_index[0].astype(jnp.int32)
  dst = edge_index[1].astype(jnp.int32)
  pad_idx = jnp.full((e_pad - e + NBUF * CHUNK,), n, dtype=jnp.int32)
  src = jnp.concatenate([src, pad_idx])
  dst = jnp.concatenate([dst, pad_idx])

  featpad = jnp.concatenate(
      [feat, jnp.ones((n, 1), jnp.float32),
       jnp.zeros((n, dw - d - 1), jnp.float32)], axis=1)
  featpad = jnp.pad(featpad, ((0, n_pad - n), (0, 0)))
  zrow = jnp.zeros((CHUNK, dw), jnp.float32)

  acc = _sc_aggregate(featpad, src, dst, zrow, n_pad, dw, ch0, ch1, rt)

  rows = 1024
  grid = -(-n // rows)
  out = pl.pallas_call(
      _tc_body,
      grid=(grid,),
      in_specs=[
          pl.BlockSpec(memory_space=pltpu.SMEM),
          pl.BlockSpec((NC, rows, dw), lambda i: (0, i, 0)),
          pl.BlockSpec((rows, d), lambda i: (i, 0)),
          pl.BlockSpec((d, d), lambda i: (0, 0)),
          pl.BlockSpec((1, d), lambda i: (0, 0)),
          pl.BlockSpec((d, d), lambda i: (0, 0)),
          pl.BlockSpec((1, d), lambda i: (0, 0)),
      ],
      out_specs=pl.BlockSpec((rows, d), lambda i: (i, 0)),
      out_shape=jax.ShapeDtypeStruct((n, d), jnp.float32),
  )(jnp.asarray(eps, jnp.float32).reshape(1, 1), acc, feat, W1,
    b1.reshape(1, d), W2, b2.reshape(1, d))
  return out


# speedup vs baseline: 2.1911x; 2.0156x over previous
import functools

import jax
import jax.numpy as jnp
from jax import lax
from jax.experimental import pallas as pl
from jax.experimental.pallas import tpu as pltpu
from jax.experimental.pallas import tpu_sc as plsc

NC = 2
NS = 16
NW = NC * NS
CHUNK = 128
NBUF = 2
CORE0_FRAC = 0.50


def _sc_aggregate(featpad, src, dst, zrow, n_pad, dw, ch0, ch1, rt):
  mesh = plsc.VectorSubcoreMesh(core_axis_name="c", subcore_axis_name="s")

  @functools.partial(
      pl.kernel,
      mesh=mesh,
      compiler_params=pltpu.CompilerParams(use_tc_tiling_on_sc=False),
      out_type=jax.ShapeDtypeStruct((NC, n_pad, dw), jnp.float32),
      scratch_types=[
          pltpu.VMEM((NBUF, CHUNK), jnp.int32),
          pltpu.VMEM((NBUF, CHUNK), jnp.int32),
          pltpu.VMEM((NBUF, CHUNK, dw), jnp.float32),
          pltpu.VMEM_SHARED((n_pad, dw), jnp.float32),
          pltpu.SemaphoreType.DMA((NBUF,)),
      ],
  )
  def sc_agg(fp_hbm, src_hbm, dst_hbm, z_hbm, out_hbm, sidx, didx, rows,
             acc, sem):
    c = lax.axis_index("c")
    s = lax.axis_index("s")
    row0 = s * rt
    for j in range(rt // CHUNK):
      pltpu.sync_copy(z_hbm, acc.at[pl.ds(row0 + j * CHUNK, CHUNK)])
    plsc.subcore_barrier()

    start_chunk = jnp.where(c == 0, s * ch0, NS * ch0 + s * ch1)
    n_outer = jnp.where(c == 0, ch0 // NBUF, ch1 // NBUF)
    ebase = start_chunk * CHUNK

    def gather_cp(b):
      return pltpu.make_async_copy(fp_hbm.at[sidx.at[b]], rows.at[b],
                                   sem.at[b])

    def prefetch(j, b):
      pltpu.sync_copy(src_hbm.at[pl.ds(ebase + j * CHUNK, CHUNK)],
                      sidx.at[b])
      pltpu.sync_copy(dst_hbm.at[pl.ds(ebase + j * CHUNK, CHUNK)],
                      didx.at[b])
      gather_cp(b).start()

    for b in range(NBUF):
      prefetch(b, b)

    def outer(i, carry):
      for b in range(NBUF):
        j = i * NBUF + b
        gather_cp(b).wait()
        pltpu.sync_copy(rows.at[b], acc.at[didx.at[b]], add=True)
        prefetch(j + NBUF, b)
      return carry

    lax.fori_loop(0, n_outer, outer, 0)
    for b in range(NBUF):
      gather_cp(b).wait()
    plsc.subcore_barrier()
    for j in range(rt // CHUNK):
      r = row0 + j * CHUNK
      pltpu.sync_copy(acc.at[pl.ds(r, CHUNK)], out_hbm.at[c, pl.ds(r, CHUNK)])

  return sc_agg(featpad, src, dst, zrow)


def _tc_body(eps_ref, acc_ref, feat_ref, w1_ref, b1_ref, w2_ref, b2_ref,
             out_ref):
  d = feat_ref.shape[1]
  s = acc_ref[0] + acc_ref[1]
  agg_sum = s[:, :d]
  deg = jnp.maximum(s[:, d:d + 1], 1.0)
  agg = agg_sum / deg
  f = feat_ref[...]
  rst = (1.0 + eps_ref[0, 0]) * f + agg
  z = jnp.dot(rst, w1_ref[...], preferred_element_type=jnp.float32)
  z = jnp.maximum(z + b1_ref[...], 0.0)
  z = jnp.dot(z, w2_ref[...], preferred_element_type=jnp.float32)
  z = jnp.maximum(z + b2_ref[...], 0.0)
  out_ref[...] = z + f


def kernel(feat, edge_index, eps, W1, b1, W2, b2):
  n, d = feat.shape
  e = edge_index.shape[1]
  dw = d + 16

  t_chunks = -(-e // (NW * CHUNK * NBUF)) * NW * NBUF
  e_pad = t_chunks * CHUNK
  ch0 = int(round(t_chunks * CORE0_FRAC / (NS * NBUF))) * NBUF
  ch1 = (t_chunks - NS * ch0) // NS
  assert ch1 % NBUF == 0 and ch1 > 0
  rt = -(-(n + 1) // (NS * CHUNK)) * CHUNK
  n_pad = NS * rt

  src = edge_index[0].astype(jnp.int32)
  dst = edge_index[1].astype(jnp.int32)
  pad_idx = n + jnp.arange(e_pad - e + NBUF * CHUNK, dtype=jnp.int32) % (
      n_pad - n)
  src = jnp.concatenate([src, pad_idx])
  dst = jnp.concatenate([dst, pad_idx])

  featpad = jnp.concatenate(
      [feat, jnp.ones((n, 1), jnp.float32),
       jnp.zeros((n, dw - d - 1), jnp.float32)], axis=1)
  featpad = jnp.pad(featpad, ((0, n_pad - n), (0, 0)))
  zrow = jnp.zeros((CHUNK, dw), jnp.float32)

  acc = _sc_aggregate(featpad, src, dst, zrow, n_pad, dw, ch0, ch1, rt)

  rows = 1024
  grid = -(-n // rows)
  out = pl.pallas_call(
      _tc_body,
      grid=(grid,),
      in_specs=[
          pl.BlockSpec(memory_space=pltpu.SMEM),
          pl.BlockSpec((NC, rows, dw), lambda i: (0, i, 0)),
          pl.BlockSpec((rows, d), lambda i: (i, 0)),
          pl.BlockSpec((d, d), lambda i: (0, 0)),
          pl.BlockSpec((1, d), lambda i: (0, 0)),
          pl.BlockSpec((d, d), lambda i: (0, 0)),
          pl.BlockSpec((1, d), lambda i: (0, 0)),
      ],
      out_specs=pl.BlockSpec((rows, d), lambda i: (i, 0)),
      out_shape=jax.ShapeDtypeStruct((n, d), jnp.float32),
  )(jnp.asarray(eps, jnp.float32).reshape(1, 1), acc, feat, W1,
    b1.reshape(1, d), W2, b2.reshape(1, d))
  return out


# speedup vs baseline: 2.5204x; 1.1503x over previous
import functools

import jax
import jax.numpy as jnp
from jax import lax
from jax.experimental import pallas as pl
from jax.experimental.pallas import tpu as pltpu
from jax.experimental.pallas import tpu_sc as plsc

NC = 2
NS = 16
NW = NC * NS
CHUNK = 128
NBUF = 2
DD = 16


def _sc_aggregate(featpad, edges, zfeat, zdeg, ones, n_pad, chunks, per_w,
                  rt, d):
  mesh = plsc.VectorSubcoreMesh(core_axis_name="c", subcore_axis_name="s")
  rpd = rt // CHUNK

  @functools.partial(
      pl.kernel,
      mesh=mesh,
      compiler_params=pltpu.CompilerParams(use_tc_tiling_on_sc=False),
      out_type=(
          jax.ShapeDtypeStruct((NC, n_pad, d), jnp.float32),
          jax.ShapeDtypeStruct((NC, n_pad, d), jnp.float32),
      ),
      scratch_types=[
          pltpu.VMEM((NBUF, CHUNK), jnp.int32),
          pltpu.VMEM((NBUF, CHUNK), jnp.int32),
          pltpu.VMEM((NBUF, CHUNK, d), jnp.float32),
          pltpu.VMEM((CHUNK, DD), jnp.float32),
          pltpu.VMEM((CHUNK, DD), jnp.float32),
          pltpu.VMEM_SHARED((n_pad, d), jnp.float32),
          pltpu.VMEM_SHARED((n_pad, DD), jnp.float32),
          pltpu.SemaphoreType.DMA((NBUF,)),
      ],
  )
  def sc_agg(fp_hbm, e_hbm, zf_hbm, zd_hbm, ones_hbm, outs_hbm, outd_hbm,
             sidx, didx, rows, onesv, degtmp, acc, dacc, sem):
    c = lax.axis_index("c")
    s = lax.axis_index("s")
    wid = c * NS + s
    row0 = s * rt
    for j in range(rpd):
      pltpu.sync_copy(zf_hbm, acc.at[pl.ds(row0 + j * CHUNK, CHUNK)])
    pltpu.sync_copy(zd_hbm, dacc.at[pl.ds(row0, rt)])
    pltpu.sync_copy(ones_hbm, onesv)
    plsc.subcore_barrier()

    ebase = wid * per_w

    def gather_cp(b):
      return pltpu.make_async_copy(fp_hbm.at[sidx.at[b]], rows.at[b],
                                   sem.at[b])

    def prefetch(j, b):
      pltpu.sync_copy(e_hbm.at[0, pl.ds(ebase + j * CHUNK, CHUNK)],
                      sidx.at[b])
      pltpu.sync_copy(e_hbm.at[1, pl.ds(ebase + j * CHUNK, CHUNK)],
                      didx.at[b])
      gather_cp(b).start()

    for b in range(NBUF):
      prefetch(b, b)

    def outer(i, carry):
      for b in range(NBUF):
        j = i * NBUF + b
        gather_cp(b).wait()
        pltpu.sync_copy(rows.at[b], acc.at[didx.at[b]], add=True)
        pltpu.sync_copy(onesv, dacc.at[didx.at[b]], add=True)
        prefetch(j + NBUF, b)
      return carry

    lax.fori_loop(0, chunks // NBUF, outer, 0)
    for b in range(NBUF):
      gather_cp(b).wait()
    plsc.subcore_barrier()
    for j in range(rpd):
      r = row0 + j * CHUNK
      pltpu.sync_copy(acc.at[pl.ds(r, CHUNK)],
                      outs_hbm.at[c, pl.ds(r, CHUNK)])
      pltpu.sync_copy(dacc.at[pl.ds(r, CHUNK)], degtmp)

      def repl(rr, carry):
        v = degtmp[rr, :]
        for k in range(d // DD):
          rows[0, rr, pl.ds(DD * k, DD)] = v
        return carry

      lax.fori_loop(0, CHUNK, repl, 0)
      pltpu.sync_copy(rows.at[0], outd_hbm.at[c, pl.ds(r, CHUNK)])

  return sc_agg(featpad, edges, zfeat, zdeg, ones)


def _tc_body(eps_ref, sums_ref, deg_ref, feat_ref, w1_ref, b1_ref, w2_ref,
             b2_ref, out_ref):
  agg_sum = sums_ref[0] + sums_ref[1]
  deg = deg_ref[0] + deg_ref[1]
  agg = agg_sum / jnp.maximum(deg, 1.0)
  f = feat_ref[...]
  rst = (1.0 + eps_ref[0, 0]) * f + agg
  z = jnp.dot(rst, w1_ref[...], preferred_element_type=jnp.float32)
  z = jnp.maximum(z + b1_ref[...], 0.0)
  z = jnp.dot(z, w2_ref[...], preferred_element_type=jnp.float32)
  z = jnp.maximum(z + b2_ref[...], 0.0)
  out_ref[...] = z + f


def kernel(feat, edge_index, eps, W1, b1, W2, b2):
  n, d = feat.shape
  e = edge_index.shape[1]

  t_chunks = -(-e // (NW * CHUNK * NBUF)) * NW * NBUF
  e_pad = t_chunks * CHUNK
  per_w = e_pad // NW
  chunks = t_chunks // NW
  rt = -(-(n + 1) // (NS * CHUNK)) * CHUNK
  n_pad = NS * rt

  npad_tail = e_pad - e + NBUF * CHUNK
  pad_idx = n + jnp.arange(npad_tail, dtype=jnp.int32) % (n_pad - n)
  edges = jnp.concatenate(
      [edge_index.astype(jnp.int32),
       jnp.broadcast_to(pad_idx, (2, npad_tail))], axis=1)

  featpad = jnp.pad(feat, ((0, n_pad - n), (0, 0)))
  zfeat = jnp.zeros((CHUNK, d), jnp.float32)
  zdeg = jnp.zeros((rt, DD), jnp.float32)
  ones = jnp.ones((CHUNK, DD), jnp.float32)

  sums, deg = _sc_aggregate(featpad, edges, zfeat, zdeg, ones, n_pad,
                            chunks, per_w, rt, d)

  rows = 1024
  grid = -(-n // rows)
  out = pl.pallas_call(
      _tc_body,
      grid=(grid,),
      in_specs=[
          pl.BlockSpec(memory_space=pltpu.SMEM),
          pl.BlockSpec((NC, rows, d), lambda i: (0, i, 0)),
          pl.BlockSpec((NC, rows, d), lambda i: (0, i, 0)),
          pl.BlockSpec((rows, d), lambda i: (i, 0)),
          pl.BlockSpec((d, d), lambda i: (0, 0)),
          pl.BlockSpec((1, d), lambda i: (0, 0)),
          pl.BlockSpec((d, d), lambda i: (0, 0)),
          pl.BlockSpec((1, d), lambda i: (0, 0)),
      ],
      out_specs=pl.BlockSpec((rows, d), lambda i: (i, 0)),
      out_shape=jax.ShapeDtypeStruct((n, d), jnp.float32),
  )(jnp.asarray(eps, jnp.float32).reshape(1, 1), sums, deg, feat, W1,
    b1.reshape(1, d), W2, b2.reshape(1, d))
  return out
